# Initial kernel scaffold; baseline (speedup 1.0000x reference)
import jax
import jax.numpy as jnp
from jax.experimental import pallas as pl


def kernel(hidden_states, sampled_indices, gamma, beta):
    raise NotImplementedError("write your pallas kernel here")


# speedup vs baseline: 3.3323x; 3.3323x over previous
import functools

import jax
import jax.numpy as jnp
from jax import lax
from jax.experimental import pallas as pl
from jax.experimental.pallas import tpu as pltpu
from jax.experimental.pallas import tpu_sc as plsc

S, B, H, R = 8192, 4, 1024, 4096
_NSEG = 8
_SEG = S // _NSEG
_BS = 256
_EPS = 1e-5
_L = 16


def _mask_body(idx_hbm, mask_hbm, idx_v, buf):
    wid = lax.axis_index("s") * 2 + lax.axis_index("c")
    b = wid // _NSEG
    j = wid % _NSEG
    base = j * _SEG
    pltpu.sync_copy(idx_hbm.at[b], idx_v)

    def _zero(i, c):
        buf[pl.ds(i * _L, _L)] = jnp.zeros((_L,), jnp.float32)
        return c

    lax.fori_loop(0, _SEG // _L, _zero, 0)

    ones = jnp.ones((_L,), jnp.float32)

    def _scatter(i, c):
        v = idx_v[pl.ds(i * _L, _L)]
        local = v - base
        inr = (local >= 0) & (local < _SEG)
        localc = jnp.clip(local, 0, _SEG - 1)
        plsc.store_scatter(buf, [localc], ones, mask=inr)
        return c

    lax.fori_loop(0, R // _L, _scatter, 0)

    pltpu.sync_copy(buf, mask_hbm.at[b, pl.ds(base, _SEG)])


@functools.cache
def _mask_fn():
    return functools.partial(
        pl.kernel,
        out_type=jax.ShapeDtypeStruct((B, S), jnp.float32),
        mesh=plsc.VectorSubcoreMesh(core_axis_name="c", subcore_axis_name="s"),
        scratch_types=[
            pltpu.VMEM((R,), jnp.int32),
            pltpu.VMEM((_SEG,), jnp.float32),
        ],
        compiler_params=pltpu.CompilerParams(needs_layout_passes=False),
    )(_mask_body)


def _ln_body(m_ref, x_ref, g_ref, bt_ref, o_ref):
    x = x_ref[...]
    mu = jnp.mean(x, axis=-1, keepdims=True)
    var = jnp.mean(jnp.square(x - mu), axis=-1, keepdims=True)
    inv = lax.rsqrt(var + _EPS)
    normed = (x - mu) * inv * g_ref[0][None, None, :] + bt_ref[0][None, None, :]
    m = m_ref[...]
    for bi in range(B):
        sel = m[bi, :][:, None] > 0.0
        o_ref[:, bi, :] = jnp.where(sel, normed[:, bi, :], x[:, bi, :])


_ln_call = pl.pallas_call(
    _ln_body,
    grid=(S // _BS,),
    in_specs=[
        pl.BlockSpec((B, _BS), lambda i: (0, i)),
        pl.BlockSpec((_BS, B, H), lambda i: (i, 0, 0)),
        pl.BlockSpec((1, H), lambda i: (0, 0)),
        pl.BlockSpec((1, H), lambda i: (0, 0)),
    ],
    out_specs=pl.BlockSpec((_BS, B, H), lambda i: (i, 0, 0)),
    out_shape=jax.ShapeDtypeStruct((S, B, H), jnp.float32),
    compiler_params=pltpu.CompilerParams(dimension_semantics=("arbitrary",)),
)


def kernel(hidden_states, sampled_indices, gamma, beta):
    idx = sampled_indices.astype(jnp.int32)
    mask = _mask_fn()(idx)
    return _ln_call(mask, hidden_states, gamma.reshape(1, H), beta.reshape(1, H))
